# baseline (speedup 1.0000x reference)
import functools

import jax
import jax.numpy as jnp
from jax import lax
from jax.experimental import pallas as pl
from jax.experimental.pallas import tpu as pltpu
from jax.experimental.pallas import tpu_sc as plsc

NC, NS, L = 2, 16, 16
NW = NC * NS
R, C = 128, 32768
RPW = R // NW
UNROLL = 8


def _scan_row_into(buf_ref, flag_ref):
    steps = C // (L * UNROLL)

    def body(k, carry):
        base = k * (L * UNROLL)
        v = buf_ref[pl.ds(base, L)]
        nan_mask = v != v
        for u in range(1, UNROLL):
            v = buf_ref[pl.ds(base + u * L, L)]
            nan_mask = jnp.logical_or(nan_mask, v != v)
        flag_ref[...] = jnp.where(nan_mask, jnp.float32(1.0), flag_ref[...])
        return carry

    lax.fori_loop(0, steps, body, 0)


def _sc_body(x_hbm, inds_hbm, out_hbm, inds_v, bufa, bufb, flag_v, sema, semb):
    wid = lax.axis_index("s") * NC + lax.axis_index("c")
    pltpu.sync_copy(inds_hbm, inds_v.at[pl.ds(0, R)])

    bufs = (bufa, bufb)
    sems = (sema, semb)

    def masked(j):
        row = wid * RPW + j
        v = inds_v[pl.ds(row, L)]
        return v[0] <= 0

    m = [masked(j) for j in range(RPW)]

    flag_v[...] = jnp.zeros((L,), jnp.float32)

    @pl.when(m[0])
    def _():
        pltpu.make_async_copy(x_hbm.at[wid * RPW], bufs[0], sems[0]).start()

    for j in range(RPW):
        if j + 1 < RPW:
            @pl.when(m[j + 1])
            def _(j=j):
                pltpu.make_async_copy(
                    x_hbm.at[wid * RPW + j + 1], bufs[(j + 1) % 2], sems[(j + 1) % 2]
                ).start()

        @pl.when(m[j])
        def _(j=j):
            pltpu.make_async_copy(
                x_hbm.at[wid * RPW + j], bufs[j % 2], sems[j % 2]
            ).wait()
            _scan_row_into(bufs[j % 2], flag_v)

    pltpu.sync_copy(flag_v, out_hbm.at[wid])


_sc_call = functools.partial(
    pl.kernel,
    out_type=jax.ShapeDtypeStruct((NW, L), jnp.float32),
    mesh=plsc.VectorSubcoreMesh(
        core_axis_name="c", subcore_axis_name="s", num_cores=NC, num_subcores=NS
    ),
    scratch_types=[
        pltpu.VMEM((R + L,), jnp.int32),
        pltpu.VMEM((C,), jnp.float32),
        pltpu.VMEM((C,), jnp.float32),
        pltpu.VMEM((L,), jnp.float32),
        pltpu.SemaphoreType.DMA,
        pltpu.SemaphoreType.DMA,
    ],
)(_sc_body)


def _tc_combine(flags_ref, out_ref):
    ok = jnp.all(flags_ref[...] == 0.0)
    out_ref[...] = jnp.where(ok, 1.0, 0.0).astype(jnp.float32) * jnp.ones(
        (1, 1), jnp.float32
    )


def kernel(x, inds):
    inds32 = jnp.asarray(inds, dtype=jnp.int32)
    flags = _sc_call(x, inds32)
    out = pl.pallas_call(
        _tc_combine,
        out_shape=jax.ShapeDtypeStruct((1, 1), jnp.float32),
    )(flags)
    return out.reshape(1)


# speedup vs baseline: 2.2493x; 2.2493x over previous
import jax
import jax.numpy as jnp
from jax.experimental import pallas as pl


def _body(inds_ref, x_ref, out_ref):
    i = pl.program_id(0)

    @pl.when(i == 0)
    def _init():
        out_ref[...] = jnp.ones((1, 1), jnp.float32)

    xb = x_ref[...]
    m1 = inds_ref[...] <= 0
    m2 = inds_ref[...] <= 0
    r1 = jnp.where(m1, xb, jnp.float32(0.0))
    r2 = jnp.where(m2, xb, jnp.float32(0.0))
    ok = jnp.all(r1 == r2)
    out_ref[...] = out_ref[...] * jnp.where(ok, 1.0, 0.0).astype(jnp.float32)


def kernel(x, inds):
    r, c = x.shape
    inds2 = jnp.asarray(inds, dtype=jnp.int32).reshape(r, 1)
    blk_r = 16
    grid = (r // blk_r,)
    out = pl.pallas_call(
        _body,
        grid=grid,
        in_specs=[
            pl.BlockSpec((blk_r, 1), lambda i: (i, 0)),
            pl.BlockSpec((blk_r, c), lambda i: (i, 0)),
        ],
        out_specs=pl.BlockSpec((1, 1), lambda i: (0, 0)),
        out_shape=jax.ShapeDtypeStruct((1, 1), jnp.float32),
    )(inds2, x)
    return out.reshape(1)


# speedup vs baseline: 2.7327x; 1.2149x over previous
import jax
import jax.numpy as jnp
from jax.experimental import pallas as pl


def _body(inds_ref, x_ref, out_ref):
    i = pl.program_id(0)

    @pl.when(i == 0)
    def _init():
        out_ref[...] = jnp.ones((1, 1), jnp.float32)

    xb = x_ref[...]
    m = inds_ref[...] <= 0
    v = jnp.where(m, xb, jnp.float32(0.0))
    ok = jnp.all(v == v)
    out_ref[...] = out_ref[...] * jnp.where(ok, 1.0, 0.0).astype(jnp.float32)


def kernel(x, inds):
    r, c = x.shape
    inds2 = jnp.asarray(inds, dtype=jnp.int32).reshape(r, 1)
    blk_c = 16384
    grid = (c // blk_c,)
    out = pl.pallas_call(
        _body,
        grid=grid,
        in_specs=[
            pl.BlockSpec((r, 1), lambda i: (0, 0)),
            pl.BlockSpec((r, blk_c), lambda i: (0, i)),
        ],
        out_specs=pl.BlockSpec((1, 1), lambda i: (0, 0)),
        out_shape=jax.ShapeDtypeStruct((1, 1), jnp.float32),
    )(inds2, x)
    return out.reshape(1)
